# baseline (speedup 1.0000x reference)
import jax
import jax.numpy as jnp
from jax import lax
from jax.experimental import pallas as pl
from jax.experimental.pallas import tpu as pltpu
from jax.experimental.pallas import tpu_sc as plsc

N_USERS = 25000
NUM_NODES = 50000
EMB_DIM = 64
H = 32
NUM_LAYERS = 3
N_EDGES = 800000

NT = 3136
NPAD = 16 * NT
FULL_CK = N_EDGES // 1024
TAIL0 = FULL_CK * 1024
NB = 64


def _body(edges, embr, usr_hbm, itm_hbm, xp_hbm,
          ridx_v, cstg_v, cidx_v, rows_v, dis_v, acc_b, out_b, ones_v,
          acc_sh, deg_sh, gs0, gs1, gs2, gs3, ss0, ss1, ss2, ss3):
    cid = lax.axis_index("c")
    sid = lax.axis_index("s")
    z16 = jnp.zeros((16,), jnp.float32)
    one16 = jnp.full((16,), 1.0, jnp.float32)
    gs = (gs0, gs1, gs2, gs3)
    ss = (ss0, ss1, ss2, ss3)

    nbase = sid * NT
    roff = cid * NPAD
    fbase = cid * H

    for j in range(8):
        ones_v[pl.ds(j * 16, 16)] = one16

    def f1(i, _):
        dis_v[pl.ds(i * 16, 16)] = z16
        return 0
    lax.fori_loop(0, NT // 16, f1, 0)

    def fz(r, _):
        out_b[r, pl.ds(0, 16)] = z16
        out_b[r, pl.ds(16, 16)] = z16
        return 0
    lax.fori_loop(0, NB, fz, 0)

    with jax.named_scope("p0_zero"):
        pltpu.sync_copy(dis_v, deg_sh.at[pl.ds(nbase, NT)])

        def z_chunk(j, _):
            pltpu.sync_copy(out_b, acc_sh.at[pl.ds(nbase + j * NB, NB)])
            return 0
        lax.fori_loop(0, NT // NB, z_chunk, 0)
        plsc.subcore_barrier()

    def load_cidx(base_e, nblk):
        pltpu.sync_copy(edges.at[1, pl.ds(base_e, nblk * 128)],
                        cstg_v.at[pl.ds(0, nblk * 128)])
        for j in range(nblk):
            for v in range(8):
                cidx_v[j, pl.ds(v * 16, 16)] = (
                    cstg_v[pl.ds((j * 8 + v) * 16, 16)])

    def load_ridx(base_e, nblk):
        pltpu.sync_copy(edges.at[0, pl.ds(base_e, nblk * 128)],
                        ridx_v.at[pl.ds(0, nblk * 128)])
        for v in range(nblk * 8):
            ridx_v[pl.ds(v * 16, 16)] = ridx_v[pl.ds(v * 16, 16)] + roff

    _p1 = jax.named_scope("p1_deg")
    _p1.__enter__()

    def deg_chunk(c, _):
        ck = c * 16 + sid

        @pl.when(ck < FULL_CK)
        def _():
            load_cidx(ck * 1024, 8)
            for j in range(8):
                pltpu.sync_copy(ones_v, deg_sh.at[cidx_v.at[j]], add=True)
        return 0
    lax.fori_loop(0, 49, deg_chunk, 0)

    @pl.when(sid == 15)
    def _():
        load_cidx(TAIL0, 2)
        for j in range(2):
            pltpu.sync_copy(ones_v, deg_sh.at[cidx_v.at[j]], add=True)
    plsc.subcore_barrier()
    _p1.__exit__(None, None, None)

    _p2 = jax.named_scope("p2_pre")
    _p2.__enter__()
    pltpu.sync_copy(deg_sh.at[pl.ds(nbase, NT)], dis_v)

    def disv(i, _):
        d = dis_v[pl.ds(i * 16, 16)]
        ii = lax.bitcast_convert_type(d, jnp.int32)
        ii = jnp.int32(0x5F3759DF) - (ii >> 1)
        y = lax.bitcast_convert_type(ii, jnp.float32)
        y = y * (1.5 - 0.5 * d * y * y)
        y = y * (1.5 - 0.5 * d * y * y)
        y = y * (1.5 - 0.5 * d * y * y)
        dis_v[pl.ds(i * 16, 16)] = jnp.where(d >= 0.5, y, z16)
        return 0
    lax.fori_loop(0, NT // 16, disv, 0)

    def out_copy(node0, nrows, to_hbm):
        def cp(hbm, r0, bb, nn):
            if to_hbm:
                pltpu.sync_copy(out_b.at[pl.ds(bb, nn)],
                                hbm.at[pl.ds(r0, nn), pl.ds(fbase, H)])
            else:
                pltpu.sync_copy(hbm.at[pl.ds(r0, nn), pl.ds(fbase, H)],
                                out_b.at[pl.ds(bb, nn)])

        @pl.when(node0 + nrows <= N_USERS)
        def _():
            cp(usr_hbm, node0, 0, nrows)

        @pl.when(node0 >= N_USERS)
        def _():
            cp(itm_hbm, node0 - N_USERS, 0, nrows)

        if nrows == NB:
            @pl.when(node0 == N_USERS - 40)
            def _():
                cp(usr_hbm, N_USERS - 40, 0, 40)
                cp(itm_hbm, 0, 40, NB - 40)

    def _prescale(j, nrows):
        node0 = nbase + j * NB
        pltpu.sync_copy(embr.at[pl.ds(node0, nrows), pl.ds(fbase, H)],
                        acc_b.at[pl.ds(0, nrows)])

        def grp(g, _):
            dvec = dis_v[pl.ds(j * NB + g * 16, 16)]
            for i in range(16):
                r = g * 16 + i
                s = dvec[i]
                e0 = acc_b[r, pl.ds(0, 16)]
                e1 = acc_b[r, pl.ds(16, 16)]
                out_b[r, pl.ds(0, 16)] = 0.5 * e0
                out_b[r, pl.ds(16, 16)] = 0.5 * e1
                acc_b[r, pl.ds(0, 16)] = s * e0
                acc_b[r, pl.ds(16, 16)] = s * e1
            return 0
        lax.fori_loop(0, nrows // 16, grp, 0)
        out_copy(node0, nrows, True)
        pltpu.sync_copy(acc_b.at[pl.ds(0, nrows)],
                        xp_hbm.at[pl.ds(roff + node0, nrows)])

    def pre_chunk(j, _):
        node0 = nbase + j * NB

        @pl.when(node0 + NB <= NUM_NODES)
        def _():
            _prescale(j, NB)

        @pl.when(jnp.logical_and(node0 < NUM_NODES,
                                 node0 + NB > NUM_NODES))
        def _():
            _prescale(j, 16)
        return 0
    lax.fori_loop(0, NT // NB, pre_chunk, 0)
    plsc.subcore_barrier()
    _p2.__exit__(None, None, None)

    for _layer in range(NUM_LAYERS):
        _pe = jax.named_scope(f"p3_edge{_layer}")
        _pe.__enter__()

        def edge_chunk(c, _):
            ck = c * 16 + sid

            @pl.when(ck < FULL_CK)
            def _():
                base_e = ck * 1024
                load_ridx(base_e, 8)
                load_cidx(base_e, 8)
                g = [pltpu.async_copy(
                        xp_hbm.at[ridx_v.at[pl.ds(j * 128, 128)]],
                        rows_v.at[j], gs[j])
                     for j in range(4)]
                s = [None] * 8
                for j in range(4):
                    g[j].wait()
                    s[j] = pltpu.async_copy(rows_v.at[j],
                                            acc_sh.at[cidx_v.at[j]],
                                            ss[j], add=True)
                g2 = [None] * 4
                for j in range(4):
                    s[j].wait()
                    g2[j] = pltpu.async_copy(
                        xp_hbm.at[ridx_v.at[pl.ds((4 + j) * 128, 128)]],
                        rows_v.at[j], gs[j])
                for j in range(4):
                    g2[j].wait()
                    s[4 + j] = pltpu.async_copy(rows_v.at[j],
                                                acc_sh.at[cidx_v.at[4 + j]],
                                                ss[j], add=True)
                for j in range(4):
                    s[4 + j].wait()
            return 0
        lax.fori_loop(0, 49, edge_chunk, 0)

        @pl.when(sid == 15)
        def _():
            load_ridx(TAIL0, 2)
            load_cidx(TAIL0, 2)
            for j in range(2):
                pltpu.async_copy(
                    xp_hbm.at[ridx_v.at[pl.ds(j * 128, 128)]],
                    rows_v.at[j], gs[j]).wait()
                pltpu.sync_copy(rows_v.at[j], acc_sh.at[cidx_v.at[j]],
                                add=True)
        plsc.subcore_barrier()
        _pe.__exit__(None, None, None)
        _pd = jax.named_scope(f"p4_drain{_layer}")
        _pd.__enter__()

        def _drain(j, nrows):
            node0 = nbase + j * NB
            pltpu.sync_copy(acc_sh.at[pl.ds(node0, nrows)],
                            acc_b.at[pl.ds(0, nrows)])
            out_copy(node0, nrows, False)

            def grp(g, _):
                dvec = dis_v[pl.ds(j * NB + g * 16, 16)]
                for i in range(16):
                    r = g * 16 + i
                    s = dvec[i]
                    a0 = acc_b[r, pl.ds(0, 16)]
                    a1 = acc_b[r, pl.ds(16, 16)]
                    x0 = s * a0
                    x1 = s * a1
                    out_b[r, pl.ds(0, 16)] = (out_b[r, pl.ds(0, 16)]
                                              + 0.5 * x0)
                    out_b[r, pl.ds(16, 16)] = (out_b[r, pl.ds(16, 16)]
                                               + 0.5 * x1)
                    acc_b[r, pl.ds(0, 16)] = s * x0
                    acc_b[r, pl.ds(16, 16)] = s * x1
                return 0
            lax.fori_loop(0, nrows // 16, grp, 0)
            out_copy(node0, nrows, True)
            pltpu.sync_copy(acc_b.at[pl.ds(0, nrows)],
                            xp_hbm.at[pl.ds(roff + node0, nrows)])
            def fz2(r, _):
                out_b[r, pl.ds(0, 16)] = z16
                out_b[r, pl.ds(16, 16)] = z16
                return 0
            lax.fori_loop(0, nrows, fz2, 0)
            pltpu.sync_copy(out_b.at[pl.ds(0, nrows)],
                            acc_sh.at[pl.ds(node0, nrows)])

        def drain_chunk(j, _):
            node0 = nbase + j * NB

            @pl.when(node0 + NB <= NUM_NODES)
            def _():
                _drain(j, NB)

            @pl.when(jnp.logical_and(node0 < NUM_NODES,
                                     node0 + NB > NUM_NODES))
            def _():
                _drain(j, 16)
            return 0
        lax.fori_loop(0, NT // NB, drain_chunk, 0)
        plsc.subcore_barrier()
        _pd.__exit__(None, None, None)


_sc_prop = pl.kernel(
    _body,
    out_type=(jax.ShapeDtypeStruct((N_USERS, EMB_DIM), jnp.float32),
              jax.ShapeDtypeStruct((NUM_NODES - N_USERS, EMB_DIM),
                                   jnp.float32),
              jax.ShapeDtypeStruct((2 * NPAD, H), jnp.float32)),
    mesh=plsc.VectorSubcoreMesh(core_axis_name="c", subcore_axis_name="s"),
    compiler_params=pltpu.CompilerParams(use_tc_tiling_on_sc=False),
    scratch_types=[
        pltpu.VMEM((1024,), jnp.int32),
        pltpu.VMEM((1024,), jnp.int32),
        pltpu.VMEM((8, 128), jnp.int32),
        pltpu.VMEM((4, 128, H), jnp.float32),
        pltpu.VMEM((NT,), jnp.float32),
        pltpu.VMEM((NB, H), jnp.float32),
        pltpu.VMEM((NB, H), jnp.float32),
        pltpu.VMEM((128,), jnp.float32),
        pltpu.VMEM_SHARED((NPAD, H), jnp.float32),
        pltpu.VMEM_SHARED((NPAD,), jnp.float32),
        pltpu.SemaphoreType.DMA,
        pltpu.SemaphoreType.DMA,
        pltpu.SemaphoreType.DMA,
        pltpu.SemaphoreType.DMA,
        pltpu.SemaphoreType.DMA,
        pltpu.SemaphoreType.DMA,
        pltpu.SemaphoreType.DMA,
        pltpu.SemaphoreType.DMA,
    ],
)


def kernel(edge_index, emb):
    usr, itm, _ = _sc_prop(edge_index.astype(jnp.int32),
                           emb.astype(jnp.float32))
    return (usr, itm)


# speedup vs baseline: 1.1082x; 1.1082x over previous
import jax
import jax.numpy as jnp
from jax import lax
from jax.experimental import pallas as pl
from jax.experimental.pallas import tpu as pltpu
from jax.experimental.pallas import tpu_sc as plsc

N_USERS = 25000
NUM_NODES = 50000
EMB_DIM = 64
H = 32
NUM_LAYERS = 3
N_EDGES = 800000

NT = 3136
NPAD = 16 * NT
FULL_CK = N_EDGES // 1024
TAIL0 = FULL_CK * 1024
NB = 64


def _body(edges, embr, usr_hbm, itm_hbm, xp_hbm,
          ridx_v, cstg_v, cidx_v, rows_v, dis_v, ones_v,
          acc_sh, deg_sh, gs0, gs1, gs2, gs3, ss0, ss1, ss2, ss3, isem):
    cid = lax.axis_index("c")
    sid = lax.axis_index("s")
    z16 = jnp.zeros((16,), jnp.float32)
    one16 = jnp.full((16,), 1.0, jnp.float32)
    gs = (gs0, gs1, gs2, gs3)
    ss = (ss0, ss1, ss2, ss3)

    nbase = sid * NT
    roff = cid * NPAD
    fbase = cid * H

    def accb(r):
        return r

    def outb(r):
        return 64 + r

    for j in range(8):
        ones_v[pl.ds(j * 16, 16)] = one16

    def f1(i, _):
        dis_v[pl.ds(i * 16, 16)] = z16
        return 0
    lax.fori_loop(0, NT // 16, f1, 0)

    def fzero_out(_=None):
        def fz(r, _):
            rows_v[3, outb(r), pl.ds(0, 16)] = z16
            rows_v[3, outb(r), pl.ds(16, 16)] = z16
            return 0
        lax.fori_loop(0, NB, fz, 0)
    fzero_out()

    with jax.named_scope("p0_zero"):
        pltpu.sync_copy(dis_v, deg_sh.at[pl.ds(nbase, NT)])

        def z_chunk(j, _):
            pltpu.sync_copy(rows_v.at[3, pl.ds(64, NB)],
                            acc_sh.at[pl.ds(nbase + j * NB, NB)])
            return 0
        lax.fori_loop(0, NT // NB, z_chunk, 0)
        plsc.subcore_barrier()

    def idx_load_descs(ck, p):
        base_e = ck * 1024
        d0 = pltpu.make_async_copy(
            edges.at[0, pl.ds(base_e, 1024)],
            ridx_v.at[pl.ds(p * 1024, 1024)], isem)
        d1 = pltpu.make_async_copy(
            edges.at[1, pl.ds(base_e, 1024)],
            cstg_v.at[pl.ds(p * 1024, 1024)], isem)
        return d0, d1

    def idx_prefetch(ck, p):
        for d in idx_load_descs(ck, p):
            d.start()

    def idx_wait(ck, p):
        for d in idx_load_descs(ck, p):
            d.wait()

    def idx_transform(p, nblk):
        for j in range(nblk):
            for v in range(8):
                cidx_v[p, j, pl.ds(v * 16, 16)] = (
                    cstg_v[pl.ds(p * 1024 + (j * 8 + v) * 16, 16)])
        for v in range(nblk * 8):
            sl = pl.ds(p * 1024 + v * 16, 16)
            ridx_v[sl] = ridx_v[sl] + roff

    with jax.named_scope("p1_deg"):
        idx_prefetch(sid, 0)

        def deg_chunk(c, _):
            ck = c * 16 + sid
            p = lax.rem(c, 2)

            @pl.when(ck < FULL_CK)
            def _():
                idx_wait(ck, p)

                @pl.when(ck + 16 < FULL_CK)
                def _():
                    idx_prefetch(ck + 16, 1 - p)
                for j in range(nblk := 8):
                    for v in range(8):
                        cidx_v[p, j, pl.ds(v * 16, 16)] = (
                            cstg_v[pl.ds(p * 1024 + (j * 8 + v) * 16, 16)])
                for j in range(8):
                    pltpu.sync_copy(ones_v, deg_sh.at[cidx_v.at[p, j]],
                                    add=True)
            return 0
        lax.fori_loop(0, 49, deg_chunk, 0)

        @pl.when(sid == 15)
        def _():
            pltpu.sync_copy(edges.at[1, pl.ds(TAIL0, 256)],
                            cstg_v.at[pl.ds(0, 256)])
            for j in range(2):
                for v in range(8):
                    cidx_v[0, j, pl.ds(v * 16, 16)] = (
                        cstg_v[pl.ds((j * 8 + v) * 16, 16)])
            for j in range(2):
                pltpu.sync_copy(ones_v, deg_sh.at[cidx_v.at[0, j]],
                                add=True)
        plsc.subcore_barrier()

    with jax.named_scope("p2_pre"):
        pltpu.sync_copy(deg_sh.at[pl.ds(nbase, NT)], dis_v)

        def disv(i, _):
            d = dis_v[pl.ds(i * 16, 16)]
            ii = lax.bitcast_convert_type(d, jnp.int32)
            ii = jnp.int32(0x5F3759DF) - (ii >> 1)
            y = lax.bitcast_convert_type(ii, jnp.float32)
            y = y * (1.5 - 0.5 * d * y * y)
            y = y * (1.5 - 0.5 * d * y * y)
            y = y * (1.5 - 0.5 * d * y * y)
            dis_v[pl.ds(i * 16, 16)] = jnp.where(d >= 0.5, y, z16)
            return 0
        lax.fori_loop(0, NT // 16, disv, 0)

        def out_copy(node0, nrows, to_hbm):
            def cp(hbm, r0, bb, nn):
                if to_hbm:
                    pltpu.sync_copy(rows_v.at[3, pl.ds(64 + bb, nn)],
                                    hbm.at[pl.ds(r0, nn), pl.ds(fbase, H)])
                else:
                    pltpu.sync_copy(hbm.at[pl.ds(r0, nn), pl.ds(fbase, H)],
                                    rows_v.at[3, pl.ds(64 + bb, nn)])

            @pl.when(node0 + nrows <= N_USERS)
            def _():
                cp(usr_hbm, node0, 0, nrows)

            @pl.when(node0 >= N_USERS)
            def _():
                cp(itm_hbm, node0 - N_USERS, 0, nrows)

            if nrows == NB:
                @pl.when(node0 == N_USERS - 40)
                def _():
                    cp(usr_hbm, N_USERS - 40, 0, 40)
                    cp(itm_hbm, 0, 40, NB - 40)

        def _prescale(j, nrows):
            node0 = nbase + j * NB
            pltpu.sync_copy(embr.at[pl.ds(node0, nrows), pl.ds(fbase, H)],
                            rows_v.at[3, pl.ds(0, nrows)])

            def grp(g, _):
                dvec = dis_v[pl.ds(j * NB + g * 16, 16)]
                for i in range(16):
                    r = g * 16 + i
                    s = dvec[i]
                    e0 = rows_v[3, accb(r), pl.ds(0, 16)]
                    e1 = rows_v[3, accb(r), pl.ds(16, 16)]
                    rows_v[3, outb(r), pl.ds(0, 16)] = 0.5 * e0
                    rows_v[3, outb(r), pl.ds(16, 16)] = 0.5 * e1
                    rows_v[3, accb(r), pl.ds(0, 16)] = s * e0
                    rows_v[3, accb(r), pl.ds(16, 16)] = s * e1
                return 0
            lax.fori_loop(0, nrows // 16, grp, 0)
            out_copy(node0, nrows, True)
            pltpu.sync_copy(rows_v.at[3, pl.ds(0, nrows)],
                            xp_hbm.at[pl.ds(roff + node0, nrows)])

        def pre_chunk(j, _):
            node0 = nbase + j * NB

            @pl.when(node0 + NB <= NUM_NODES)
            def _():
                _prescale(j, NB)

            @pl.when(jnp.logical_and(node0 < NUM_NODES,
                                     node0 + NB > NUM_NODES))
            def _():
                _prescale(j, 16)
            return 0
        lax.fori_loop(0, NT // NB, pre_chunk, 0)
        plsc.subcore_barrier()

    for _layer in range(NUM_LAYERS):
        _pe = jax.named_scope(f"p3_edge{_layer}")
        _pe.__enter__()
        idx_prefetch(sid, 0)

        def edge_chunk(c, _):
            ck = c * 16 + sid
            p = lax.rem(c, 2)

            @pl.when(ck < FULL_CK)
            def _():
                idx_wait(ck, p)

                @pl.when(ck + 16 < FULL_CK)
                def _():
                    idx_prefetch(ck + 16, 1 - p)
                idx_transform(p, 8)

                def gblk(j):
                    return pltpu.async_copy(
                        xp_hbm.at[ridx_v.at[pl.ds(p * 1024 + j * 128,
                                                  128)]],
                        rows_v.at[j % 4], gs[j % 4])

                g = [gblk(j) for j in range(4)]
                s = [None] * 8
                for j in range(4):
                    g[j].wait()
                    s[j] = pltpu.async_copy(rows_v.at[j],
                                            acc_sh.at[cidx_v.at[p, j]],
                                            ss[j], add=True)
                g2 = [None] * 4
                for j in range(4):
                    s[j].wait()
                    g2[j] = gblk(4 + j)
                for j in range(4):
                    g2[j].wait()
                    s[4 + j] = pltpu.async_copy(
                        rows_v.at[j], acc_sh.at[cidx_v.at[p, 4 + j]],
                        ss[j], add=True)
                for j in range(4):
                    s[4 + j].wait()
            return 0
        lax.fori_loop(0, 49, edge_chunk, 0)

        @pl.when(sid == 15)
        def _():
            pltpu.sync_copy(edges.at[0, pl.ds(TAIL0, 256)],
                            ridx_v.at[pl.ds(0, 256)])
            pltpu.sync_copy(edges.at[1, pl.ds(TAIL0, 256)],
                            cstg_v.at[pl.ds(0, 256)])
            for j in range(2):
                for v in range(8):
                    cidx_v[0, j, pl.ds(v * 16, 16)] = (
                        cstg_v[pl.ds((j * 8 + v) * 16, 16)])
            for v in range(16):
                sl = pl.ds(v * 16, 16)
                ridx_v[sl] = ridx_v[sl] + roff
            for j in range(2):
                pltpu.async_copy(
                    xp_hbm.at[ridx_v.at[pl.ds(j * 128, 128)]],
                    rows_v.at[j], gs[j]).wait()
                pltpu.sync_copy(rows_v.at[j], acc_sh.at[cidx_v.at[0, j]],
                                add=True)
        plsc.subcore_barrier()
        _pe.__exit__(None, None, None)
        _pd = jax.named_scope(f"p4_drain{_layer}")
        _pd.__enter__()

        def _drain(j, nrows):
            node0 = nbase + j * NB
            pltpu.sync_copy(acc_sh.at[pl.ds(node0, nrows)],
                            rows_v.at[3, pl.ds(0, nrows)])
            out_copy(node0, nrows, False)

            def grp(g, _):
                dvec = dis_v[pl.ds(j * NB + g * 16, 16)]
                for i in range(16):
                    r = g * 16 + i
                    s = dvec[i]
                    a0 = rows_v[3, accb(r), pl.ds(0, 16)]
                    a1 = rows_v[3, accb(r), pl.ds(16, 16)]
                    x0 = s * a0
                    x1 = s * a1
                    rows_v[3, outb(r), pl.ds(0, 16)] = (
                        rows_v[3, outb(r), pl.ds(0, 16)] + 0.5 * x0)
                    rows_v[3, outb(r), pl.ds(16, 16)] = (
                        rows_v[3, outb(r), pl.ds(16, 16)] + 0.5 * x1)
                    rows_v[3, accb(r), pl.ds(0, 16)] = s * x0
                    rows_v[3, accb(r), pl.ds(16, 16)] = s * x1
                return 0
            lax.fori_loop(0, nrows // 16, grp, 0)
            out_copy(node0, nrows, True)
            pltpu.sync_copy(rows_v.at[3, pl.ds(0, nrows)],
                            xp_hbm.at[pl.ds(roff + node0, nrows)])
            def fz2(r, _):
                rows_v[3, outb(r), pl.ds(0, 16)] = z16
                rows_v[3, outb(r), pl.ds(16, 16)] = z16
                return 0
            lax.fori_loop(0, nrows, fz2, 0)
            pltpu.sync_copy(rows_v.at[3, pl.ds(64, nrows)],
                            acc_sh.at[pl.ds(node0, nrows)])

        def drain_chunk(j, _):
            node0 = nbase + j * NB

            @pl.when(node0 + NB <= NUM_NODES)
            def _():
                _drain(j, NB)

            @pl.when(jnp.logical_and(node0 < NUM_NODES,
                                     node0 + NB > NUM_NODES))
            def _():
                _drain(j, 16)
            return 0
        lax.fori_loop(0, NT // NB, drain_chunk, 0)
        plsc.subcore_barrier()
        _pd.__exit__(None, None, None)


_sc_prop = pl.kernel(
    _body,
    out_type=(jax.ShapeDtypeStruct((N_USERS, EMB_DIM), jnp.float32),
              jax.ShapeDtypeStruct((NUM_NODES - N_USERS, EMB_DIM),
                                   jnp.float32),
              jax.ShapeDtypeStruct((2 * NPAD, H), jnp.float32)),
    mesh=plsc.VectorSubcoreMesh(core_axis_name="c", subcore_axis_name="s"),
    compiler_params=pltpu.CompilerParams(use_tc_tiling_on_sc=False),
    scratch_types=[
        pltpu.VMEM((2048,), jnp.int32),
        pltpu.VMEM((2048,), jnp.int32),
        pltpu.VMEM((2, 8, 128), jnp.int32),
        pltpu.VMEM((4, 128, H), jnp.float32),
        pltpu.VMEM((NT,), jnp.float32),
        pltpu.VMEM((128,), jnp.float32),
        pltpu.VMEM_SHARED((NPAD, H), jnp.float32),
        pltpu.VMEM_SHARED((NPAD,), jnp.float32),
        pltpu.SemaphoreType.DMA,
        pltpu.SemaphoreType.DMA,
        pltpu.SemaphoreType.DMA,
        pltpu.SemaphoreType.DMA,
        pltpu.SemaphoreType.DMA,
        pltpu.SemaphoreType.DMA,
        pltpu.SemaphoreType.DMA,
        pltpu.SemaphoreType.DMA,
        pltpu.SemaphoreType.DMA,
    ],
)


def kernel(edge_index, emb):
    usr, itm, _ = _sc_prop(edge_index.astype(jnp.int32),
                           emb.astype(jnp.float32))
    return (usr, itm)


# speedup vs baseline: 1.2119x; 1.0936x over previous
import jax
import jax.numpy as jnp
from jax import lax
from jax.experimental import pallas as pl
from jax.experimental.pallas import tpu as pltpu
from jax.experimental.pallas import tpu_sc as plsc

N_USERS = 25000
NUM_NODES = 50000
EMB_DIM = 64
H = 32
NUM_LAYERS = 3
N_EDGES = 800000

NT = 3136
NPAD = 16 * NT
FULL_CK = N_EDGES // 1024
TAIL0 = FULL_CK * 1024
NB = 64


def _body(edges, embr, usr_hbm, itm_hbm, xp_hbm,
          ridx_v, cstg_v, cidx_v, rows_v, dis_v, ones_v,
          acc_sh, deg_sh, gs0, gs1, gs2, gs3, ss0, ss1, ss2, ss3, isem,
          wsem):
    cid = lax.axis_index("c")
    sid = lax.axis_index("s")
    z16 = jnp.zeros((16,), jnp.float32)
    one16 = jnp.full((16,), 1.0, jnp.float32)
    gs = (gs0, gs1, gs2, gs3)
    ss = (ss0, ss1, ss2, ss3)

    nbase = sid * NT
    roff = cid * NPAD
    fbase = cid * H

    def accb(r):
        return r

    def outb(r):
        return 64 + r

    for j in range(8):
        ones_v[pl.ds(j * 16, 16)] = one16

    def f1(i, _):
        dis_v[pl.ds(i * 16, 16)] = z16
        return 0
    lax.fori_loop(0, NT // 16, f1, 0)

    def fzero_out(_=None):
        def fz(r, _):
            rows_v[3, outb(r), pl.ds(0, 16)] = z16
            rows_v[3, outb(r), pl.ds(16, 16)] = z16
            return 0
        lax.fori_loop(0, NB, fz, 0)
    fzero_out()

    with jax.named_scope("p0_zero"):
        pltpu.sync_copy(dis_v, deg_sh.at[pl.ds(nbase, NT)])

        def z_chunk(j, _):
            pltpu.sync_copy(rows_v.at[3, pl.ds(64, NB)],
                            acc_sh.at[pl.ds(nbase + j * NB, NB)])
            return 0
        lax.fori_loop(0, NT // NB, z_chunk, 0)
        plsc.subcore_barrier()

    def idx_load_descs(ck, p):
        base_e = ck * 1024
        d0 = pltpu.make_async_copy(
            edges.at[0, pl.ds(base_e, 1024)],
            ridx_v.at[pl.ds(p * 1024, 1024)], isem)
        d1 = pltpu.make_async_copy(
            edges.at[1, pl.ds(base_e, 1024)],
            cstg_v.at[pl.ds(p * 1024, 1024)], isem)
        return d0, d1

    def idx_prefetch(ck, p):
        for d in idx_load_descs(ck, p):
            d.start()

    def idx_wait(ck, p):
        for d in idx_load_descs(ck, p):
            d.wait()

    def idx_transform(p, nblk):
        for j in range(nblk):
            for v in range(8):
                cidx_v[p, j, pl.ds(v * 16, 16)] = (
                    cstg_v[pl.ds(p * 1024 + (j * 8 + v) * 16, 16)])
        for v in range(nblk * 8):
            sl = pl.ds(p * 1024 + v * 16, 16)
            ridx_v[sl] = ridx_v[sl] + roff

    with jax.named_scope("p1_deg"):
        idx_prefetch(sid, 0)

        def deg_chunk(c, _):
            ck = c * 16 + sid
            p = lax.rem(c, 2)

            @pl.when(ck < FULL_CK)
            def _():
                idx_wait(ck, p)

                @pl.when(ck + 16 < FULL_CK)
                def _():
                    idx_prefetch(ck + 16, 1 - p)
                for j in range(nblk := 8):
                    for v in range(8):
                        cidx_v[p, j, pl.ds(v * 16, 16)] = (
                            cstg_v[pl.ds(p * 1024 + (j * 8 + v) * 16, 16)])
                for j in range(8):
                    pltpu.sync_copy(ones_v, deg_sh.at[cidx_v.at[p, j]],
                                    add=True)
            return 0
        lax.fori_loop(0, 49, deg_chunk, 0)

        @pl.when(sid == 15)
        def _():
            pltpu.sync_copy(edges.at[1, pl.ds(TAIL0, 256)],
                            cstg_v.at[pl.ds(0, 256)])
            for j in range(2):
                for v in range(8):
                    cidx_v[0, j, pl.ds(v * 16, 16)] = (
                        cstg_v[pl.ds((j * 8 + v) * 16, 16)])
            for j in range(2):
                pltpu.sync_copy(ones_v, deg_sh.at[cidx_v.at[0, j]],
                                add=True)
        plsc.subcore_barrier()

    with jax.named_scope("p2_pre"):
        pltpu.sync_copy(deg_sh.at[pl.ds(nbase, NT)], dis_v)

        def disv(i, _):
            d = dis_v[pl.ds(i * 16, 16)]
            ii = lax.bitcast_convert_type(d, jnp.int32)
            ii = jnp.int32(0x5F3759DF) - (ii >> 1)
            y = lax.bitcast_convert_type(ii, jnp.float32)
            y = y * (1.5 - 0.5 * d * y * y)
            y = y * (1.5 - 0.5 * d * y * y)
            y = y * (1.5 - 0.5 * d * y * y)
            dis_v[pl.ds(i * 16, 16)] = jnp.where(d >= 0.5, y, z16)
            return 0
        lax.fori_loop(0, NT // 16, disv, 0)

        def out_copy(node0, nrows, to_hbm):
            def cp(hbm, r0, bb, nn):
                if to_hbm:
                    pltpu.sync_copy(rows_v.at[3, pl.ds(64 + bb, nn)],
                                    hbm.at[pl.ds(r0, nn), pl.ds(fbase, H)])
                else:
                    pltpu.sync_copy(hbm.at[pl.ds(r0, nn), pl.ds(fbase, H)],
                                    rows_v.at[3, pl.ds(64 + bb, nn)])

            @pl.when(node0 + nrows <= N_USERS)
            def _():
                cp(usr_hbm, node0, 0, nrows)

            @pl.when(node0 >= N_USERS)
            def _():
                cp(itm_hbm, node0 - N_USERS, 0, nrows)

            if nrows == NB:
                @pl.when(node0 == N_USERS - 40)
                def _():
                    cp(usr_hbm, N_USERS - 40, 0, 40)
                    cp(itm_hbm, 0, 40, NB - 40)

        def out_wdescs(node0, nrows):
            def mk(hbm, r0, bb, nn):
                return pltpu.make_async_copy(
                    rows_v.at[3, pl.ds(64 + bb, nn)],
                    hbm.at[pl.ds(r0, nn), pl.ds(fbase, H)], wsem)

            def run(start):
                def go(d):
                    d.start() if start else d.wait()

                @pl.when(node0 + nrows <= N_USERS)
                def _():
                    go(mk(usr_hbm, node0, 0, nrows))

                @pl.when(node0 >= N_USERS)
                def _():
                    go(mk(itm_hbm, node0 - N_USERS, 0, nrows))

                if nrows == NB:
                    @pl.when(node0 == N_USERS - 40)
                    def _():
                        go(mk(usr_hbm, N_USERS - 40, 0, 40))
                        go(mk(itm_hbm, 0, 40, NB - 40))
            run(True)
            return lambda: run(False)

        def _prescale(j, nrows):
            node0 = nbase + j * NB
            pltpu.sync_copy(embr.at[pl.ds(node0, nrows), pl.ds(fbase, H)],
                            rows_v.at[3, pl.ds(0, nrows)])

            def grp(g, _):
                dvec = dis_v[pl.ds(j * NB + g * 16, 16)]
                for i in range(16):
                    r = g * 16 + i
                    s = dvec[i]
                    e0 = rows_v[3, accb(r), pl.ds(0, 16)]
                    e1 = rows_v[3, accb(r), pl.ds(16, 16)]
                    rows_v[3, outb(r), pl.ds(0, 16)] = 0.5 * e0
                    rows_v[3, outb(r), pl.ds(16, 16)] = 0.5 * e1
                    rows_v[3, accb(r), pl.ds(0, 16)] = s * e0
                    rows_v[3, accb(r), pl.ds(16, 16)] = s * e1
                return 0
            lax.fori_loop(0, nrows // 16, grp, 0)
            wd = out_wdescs(node0, nrows)
            xd = pltpu.async_copy(rows_v.at[3, pl.ds(0, nrows)],
                                  xp_hbm.at[pl.ds(roff + node0, nrows)],
                                  isem)
            wd()
            xd.wait()

        def pre_chunk(j, _):
            node0 = nbase + j * NB

            @pl.when(node0 + NB <= NUM_NODES)
            def _():
                _prescale(j, NB)

            @pl.when(jnp.logical_and(node0 < NUM_NODES,
                                     node0 + NB > NUM_NODES))
            def _():
                _prescale(j, 16)
            return 0
        lax.fori_loop(0, NT // NB, pre_chunk, 0)
        plsc.subcore_barrier()

    for _layer in range(NUM_LAYERS):
        _pe = jax.named_scope(f"p3_edge{_layer}")
        _pe.__enter__()
        idx_prefetch(sid, 0)

        def edge_chunk(c, _):
            ck = c * 16 + sid
            p = lax.rem(c, 2)

            @pl.when(ck < FULL_CK)
            def _():
                idx_wait(ck, p)

                @pl.when(ck + 16 < FULL_CK)
                def _():
                    idx_prefetch(ck + 16, 1 - p)
                idx_transform(p, 8)

                def gblk(j):
                    return pltpu.async_copy(
                        xp_hbm.at[ridx_v.at[pl.ds(p * 1024 + j * 128,
                                                  128)]],
                        rows_v.at[j % 4], gs[j % 4])

                g = [gblk(j) for j in range(4)]
                s = [None] * 8
                for j in range(4):
                    g[j].wait()
                    s[j] = pltpu.async_copy(rows_v.at[j],
                                            acc_sh.at[cidx_v.at[p, j]],
                                            ss[j], add=True)
                g2 = [None] * 4
                for j in range(4):
                    s[j].wait()
                    g2[j] = gblk(4 + j)
                for j in range(4):
                    g2[j].wait()
                    s[4 + j] = pltpu.async_copy(
                        rows_v.at[j], acc_sh.at[cidx_v.at[p, 4 + j]],
                        ss[j], add=True)
                for j in range(4):
                    s[4 + j].wait()
            return 0
        lax.fori_loop(0, 49, edge_chunk, 0)

        @pl.when(sid == 15)
        def _():
            pltpu.sync_copy(edges.at[0, pl.ds(TAIL0, 256)],
                            ridx_v.at[pl.ds(0, 256)])
            pltpu.sync_copy(edges.at[1, pl.ds(TAIL0, 256)],
                            cstg_v.at[pl.ds(0, 256)])
            for j in range(2):
                for v in range(8):
                    cidx_v[0, j, pl.ds(v * 16, 16)] = (
                        cstg_v[pl.ds((j * 8 + v) * 16, 16)])
            for v in range(16):
                sl = pl.ds(v * 16, 16)
                ridx_v[sl] = ridx_v[sl] + roff
            for j in range(2):
                pltpu.async_copy(
                    xp_hbm.at[ridx_v.at[pl.ds(j * 128, 128)]],
                    rows_v.at[j], gs[j]).wait()
                pltpu.sync_copy(rows_v.at[j], acc_sh.at[cidx_v.at[0, j]],
                                add=True)
        plsc.subcore_barrier()
        _pe.__exit__(None, None, None)
        _pd = jax.named_scope(f"p4_drain{_layer}")
        _pd.__enter__()

        def fzs(r, _):
            rows_v[2, r, pl.ds(0, 16)] = z16
            rows_v[2, r, pl.ds(16, 16)] = z16
            return 0
        lax.fori_loop(0, NB, fzs, 0)

        def _drain(j, nrows):
            node0 = nbase + j * NB
            ad = pltpu.async_copy(acc_sh.at[pl.ds(node0, nrows)],
                                  rows_v.at[3, pl.ds(0, nrows)], wsem)
            out_copy(node0, nrows, False)
            ad.wait()

            def grp(g, _):
                dvec = dis_v[pl.ds(j * NB + g * 16, 16)]
                for i in range(16):
                    r = g * 16 + i
                    s = dvec[i]
                    a0 = rows_v[3, accb(r), pl.ds(0, 16)]
                    a1 = rows_v[3, accb(r), pl.ds(16, 16)]
                    x0 = s * a0
                    x1 = s * a1
                    rows_v[3, outb(r), pl.ds(0, 16)] = (
                        rows_v[3, outb(r), pl.ds(0, 16)] + 0.5 * x0)
                    rows_v[3, outb(r), pl.ds(16, 16)] = (
                        rows_v[3, outb(r), pl.ds(16, 16)] + 0.5 * x1)
                    rows_v[3, accb(r), pl.ds(0, 16)] = s * x0
                    rows_v[3, accb(r), pl.ds(16, 16)] = s * x1
                return 0
            lax.fori_loop(0, nrows // 16, grp, 0)
            wd = out_wdescs(node0, nrows)
            xd = pltpu.async_copy(rows_v.at[3, pl.ds(0, nrows)],
                                  xp_hbm.at[pl.ds(roff + node0, nrows)],
                                  isem)
            zd = pltpu.async_copy(rows_v.at[2, pl.ds(0, nrows)],
                                  acc_sh.at[pl.ds(node0, nrows)], gs0)
            wd()
            xd.wait()
            zd.wait()

        def drain_chunk(j, _):
            node0 = nbase + j * NB

            @pl.when(node0 + NB <= NUM_NODES)
            def _():
                _drain(j, NB)

            @pl.when(jnp.logical_and(node0 < NUM_NODES,
                                     node0 + NB > NUM_NODES))
            def _():
                _drain(j, 16)
            return 0
        lax.fori_loop(0, NT // NB, drain_chunk, 0)
        plsc.subcore_barrier()
        _pd.__exit__(None, None, None)


_sc_prop = pl.kernel(
    _body,
    out_type=(jax.ShapeDtypeStruct((N_USERS, EMB_DIM), jnp.float32),
              jax.ShapeDtypeStruct((NUM_NODES - N_USERS, EMB_DIM),
                                   jnp.float32),
              jax.ShapeDtypeStruct((2 * NPAD, H), jnp.float32)),
    mesh=plsc.VectorSubcoreMesh(core_axis_name="c", subcore_axis_name="s"),
    compiler_params=pltpu.CompilerParams(use_tc_tiling_on_sc=False),
    scratch_types=[
        pltpu.VMEM((2048,), jnp.int32),
        pltpu.VMEM((2048,), jnp.int32),
        pltpu.VMEM((2, 8, 128), jnp.int32),
        pltpu.VMEM((4, 128, H), jnp.float32),
        pltpu.VMEM((NT,), jnp.float32),
        pltpu.VMEM((128,), jnp.float32),
        pltpu.VMEM_SHARED((NPAD, H), jnp.float32),
        pltpu.VMEM_SHARED((NPAD,), jnp.float32),
        pltpu.SemaphoreType.DMA,
        pltpu.SemaphoreType.DMA,
        pltpu.SemaphoreType.DMA,
        pltpu.SemaphoreType.DMA,
        pltpu.SemaphoreType.DMA,
        pltpu.SemaphoreType.DMA,
        pltpu.SemaphoreType.DMA,
        pltpu.SemaphoreType.DMA,
        pltpu.SemaphoreType.DMA,
        pltpu.SemaphoreType.DMA,
    ],
)


def kernel(edge_index, emb):
    usr, itm, _ = _sc_prop(edge_index.astype(jnp.int32),
                           emb.astype(jnp.float32))
    return (usr, itm)
